# baseline (speedup 1.0000x reference)
import functools

import jax
import jax.numpy as jnp
from jax import lax
from jax.experimental import pallas as pl
from jax.experimental.pallas import tpu as pltpu
from jax.experimental.pallas import tpu_sc as plsc

NH = 16
MAXD = 32
S = 2048
W = 2 * MAXD + 1
MPAD = 4224
K = 16


def _build_body(table_ref, m8_ref):
    u = jax.lax.broadcasted_iota(jnp.int32, (8, MPAD), 1)
    p = jax.lax.broadcasted_iota(jnp.int32, (8, MPAD), 0)
    idx = jnp.clip(u + p - S, -MAXD, MAXD) + MAXD
    acc = jnp.full((8, MPAD), table_ref[0, 0, 0], dtype=jnp.float32)
    for k in range(1, W):
        acc = jnp.where(idx == k, table_ref[0, 0, k], acc)
    m8_ref[0] = acc


def _build_m8(relative_bias):
    return pl.pallas_call(
        _build_body,
        grid=(NH,),
        in_specs=[
            pl.BlockSpec((1, 1, W), lambda h: (h, 0, 0),
                         memory_space=pltpu.SMEM),
        ],
        out_specs=pl.BlockSpec((1, 8, MPAD), lambda h: (h, 0, 0)),
        out_shape=jax.ShapeDtypeStruct((NH, 8, MPAD), jnp.float32),
    )(relative_bias.reshape(NH, 1, W))


def _sc_materialize(m8):
    info = plsc.get_sparse_core_info()
    nc, ns = info.num_cores, info.num_subcores
    rows_per_w = NH * S // (nc * ns)
    mesh = plsc.VectorSubcoreMesh(core_axis_name="c", subcore_axis_name="s")

    @functools.partial(
        pl.kernel,
        mesh=mesh,
        out_type=jax.ShapeDtypeStruct((NH * S * S,), jnp.float32),
        scratch_types=[
            pltpu.VMEM((8 * MPAD,), jnp.float32),
            pltpu.SemaphoreType.DMA,
        ],
    )
    def sc_k(m8_hbm, out_hbm, m_v, sem):
        wid = lax.axis_index("s") * nc + lax.axis_index("c")
        h = wid // (S // rows_per_w)
        i0 = (wid % (S // rows_per_w)) * rows_per_w
        pltpu.sync_copy(m8_hbm.at[h], m_v)

        def row_copy(i):
            start = S - i
            p = lax.rem(start, 8)
            q = start - p
            off = (h * S + i) * S
            return pltpu.make_async_copy(
                m_v.at[pl.ds(pl.multiple_of(p * MPAD + q, 8), S)],
                out_hbm.at[pl.ds(pl.multiple_of(off, 128), S)],
                sem,
            )

        def chunk(g, carry):
            for j in range(K):
                row_copy(i0 + g * K + j).start()

            @pl.when(g > 0)
            def _drain():
                for j in range(K):
                    row_copy(i0 + (g - 1) * K + j).wait()

            return carry

        nchunks = rows_per_w // K
        lax.fori_loop(0, nchunks, chunk, 0)
        for j in range(K):
            row_copy(i0 + (nchunks - 1) * K + j).wait()

    return sc_k(m8)


def kernel(seq_len, relative_bias):
    del seq_len
    m8 = _build_m8(relative_bias).reshape(NH, 8 * MPAD)
    return _sc_materialize(m8).reshape(NH, S, S)
